# Initial kernel scaffold; baseline (speedup 1.0000x reference)
import jax
import jax.numpy as jnp
from jax.experimental import pallas as pl


def kernel(x, conv1_w, conv1_b, rw1_w, rw1_b, conv2_w, conv2_b, rw2_w, rw2_b, codebook):
    raise NotImplementedError("write your pallas kernel here")


# speedup vs baseline: 1.0216x; 1.0216x over previous
import jax
import jax.numpy as jnp
from jax.experimental import pallas as pl


_F32 = jnp.float32


def _enc1_kernel(p1_ref, w1_ref, b1_ref, rw1_ref, rb1_ref, out_ref):
    p1 = p1_ref[0]
    y = jnp.dot(w1_ref[...], p1, preferred_element_type=_F32) + b1_ref[...]
    y = jax.nn.gelu(y)
    z = jnp.dot(rw1_ref[...], y, preferred_element_type=_F32) + rb1_ref[...]
    out_ref[0] = z[:48] * jax.nn.sigmoid(z[48:])


def _enc2_vq_kernel(p2_ref, w2_ref, b2_ref, rw2_ref, rb2_ref, cb_ref, cbt_ref,
                    bvq_ref, lat_ref, idx_ref):
    p2 = p2_ref[0]
    y = jnp.dot(w2_ref[...], p2, preferred_element_type=_F32) + b2_ref[...]
    y = jax.nn.gelu(y)
    z = jnp.dot(rw2_ref[...], y, preferred_element_type=_F32) + rb2_ref[...]
    g = z[:96] * jax.nn.sigmoid(z[96:])
    bvq_ref[0] = g

    cb = cb_ref[...]
    scores = jnp.dot(cb, g, preferred_element_type=_F32)
    fsq = jnp.sum(g * g, axis=0, keepdims=True)
    cbsq = jnp.sum(cb * cb, axis=1, keepdims=True)
    dist = (fsq - 2.0 * scores) + cbsq
    minval = jnp.min(dist, axis=0, keepdims=True)
    kiota = jax.lax.broadcasted_iota(jnp.int32, dist.shape, 0)
    idx = jnp.min(jnp.where(dist == minval, kiota, 1024), axis=0, keepdims=True)
    idx_ref[0, 0] = idx

    onehot = (kiota == idx).astype(_F32)
    quant = jnp.dot(cbt_ref[...], onehot, preferred_element_type=_F32)
    lat_ref[0] = g + (quant - g)


def kernel(x, conv1_w, conv1_b, rw1_w, rw1_b, conv2_w, conv2_b, rw2_w, rw2_b,
           codebook):
    B = x.shape[0]
    N1 = 128 * 256
    N2 = 32 * 256
    S1 = 16384
    S2 = 2048
    NT1 = N1 // S1
    NT2 = N2 // S2

    xp = jnp.pad(x, ((0, 0), (0, 0), (2, 2), (0, 0)))
    p1 = jnp.stack([xp[:, :, kh:kh + 509:4, :] for kh in range(8)], axis=1)
    p1 = p1.reshape(B, 16, N1)
    w1m = jnp.transpose(conv1_w[:, :, :, 0], (0, 2, 1)).reshape(48, 16)
    rw1m = rw1_w[:, :, 0, 0]

    glu1 = pl.pallas_call(
        _enc1_kernel,
        grid=(B, NT1),
        in_specs=[
            pl.BlockSpec((1, 16, S1), lambda b, t: (b, 0, t)),
            pl.BlockSpec((48, 16), lambda b, t: (0, 0)),
            pl.BlockSpec((48, 1), lambda b, t: (0, 0)),
            pl.BlockSpec((96, 48), lambda b, t: (0, 0)),
            pl.BlockSpec((96, 1), lambda b, t: (0, 0)),
        ],
        out_specs=pl.BlockSpec((1, 48, S1), lambda b, t: (b, 0, t)),
        out_shape=jax.ShapeDtypeStruct((B, 48, N1), _F32),
    )(p1, w1m, conv1_b[:, None], rw1m, rw1_b[:, None])

    gp = jnp.pad(glu1.reshape(B, 48, 128, 256), ((0, 0), (0, 0), (2, 2), (0, 0)))
    p2 = jnp.stack([gp[:, :, kh:kh + 125:4, :] for kh in range(8)], axis=1)
    p2 = p2.reshape(B, 384, N2)
    w2m = jnp.transpose(conv2_w[:, :, :, 0], (0, 2, 1)).reshape(96, 384)
    rw2m = rw2_w[:, :, 0, 0]

    bvq, lat, idx = pl.pallas_call(
        _enc2_vq_kernel,
        grid=(B, NT2),
        in_specs=[
            pl.BlockSpec((1, 384, S2), lambda b, t: (b, 0, t)),
            pl.BlockSpec((96, 384), lambda b, t: (0, 0)),
            pl.BlockSpec((96, 1), lambda b, t: (0, 0)),
            pl.BlockSpec((192, 96), lambda b, t: (0, 0)),
            pl.BlockSpec((192, 1), lambda b, t: (0, 0)),
            pl.BlockSpec((1024, 96), lambda b, t: (0, 0)),
            pl.BlockSpec((96, 1024), lambda b, t: (0, 0)),
        ],
        out_specs=[
            pl.BlockSpec((1, 96, S2), lambda b, t: (b, 0, t)),
            pl.BlockSpec((1, 96, S2), lambda b, t: (b, 0, t)),
            pl.BlockSpec((1, 1, 1, S2), lambda b, t: (b, t, 0, 0)),
        ],
        out_shape=[
            jax.ShapeDtypeStruct((B, 96, N2), _F32),
            jax.ShapeDtypeStruct((B, 96, N2), _F32),
            jax.ShapeDtypeStruct((B, NT2, 1, S2), jnp.int32),
        ],
    )(p2, w2m, conv2_b[:, None], rw2m, rw2_b[:, None], codebook,
      jnp.transpose(codebook))

    latent = lat.reshape(B, 96, 32, 256)
    beforvq = bvq.reshape(B, 96, 32, 256)
    indices = idx.reshape(B, N2)
    return (latent, indices, beforvq)


# speedup vs baseline: 2.4966x; 2.4438x over previous
import numpy as np

import jax
import jax.numpy as jnp
from jax.experimental import pallas as pl
from jax.experimental.pallas import tpu as pltpu


_F32 = jnp.float32
_SEC = 8448
_S2 = 2048
_NT2 = 4

_H1_ORDER = np.concatenate([
    np.arange(1, 33) * 4 - 2,
    np.arange(1, 33) * 4 - 1,
    np.arange(0, 32) * 4,
    np.arange(0, 32) * 4 + 1,
])
_PAD_CHUNKS = (0, _SEC, 2 * _SEC + 8192, 3 * _SEC + 8192)


def _fused_kernel(p1_ref, w1_ref, b1_ref, rw1_ref, rb1_ref,
                  w2_ref, b2_ref, rw2_ref, rb2_ref, cb_ref, cbt_ref,
                  bvq_ref, lat_ref, idx_ref, gscr):
    for col in _PAD_CHUNKS:
        gscr[:, col:col + 256] = jnp.zeros((48, 256), _F32)
    for r in range(4):
        p1s = p1_ref[0, :, 8192 * r:8192 * (r + 1)]
        y = jnp.dot(w1_ref[...], p1s, preferred_element_type=_F32) + b1_ref[...]
        y = jax.nn.gelu(y)
        z = jnp.dot(rw1_ref[...], y, preferred_element_type=_F32) + rb1_ref[...]
        g = z[:48] * jax.nn.sigmoid(z[48:])
        off = _SEC * r + (256 if r < 2 else 0)
        gscr[:, off:off + 8192] = g

    cb = cb_ref[...]
    cbsq = jnp.sum(cb * cb, axis=1, keepdims=True)
    for t in range(_NT2):
        pieces = []
        for kh in range(8):
            q, r = kh // 4, kh % 4
            off = _SEC * r + 256 * q + _S2 * t
            pieces.append(gscr[:, off:off + _S2])
        p2 = jnp.concatenate(pieces, axis=0)
        y = jnp.dot(w2_ref[...], p2, preferred_element_type=_F32) + b2_ref[...]
        y = jax.nn.gelu(y)
        z = jnp.dot(rw2_ref[...], y, preferred_element_type=_F32) + rb2_ref[...]
        g = z[:96] * jax.nn.sigmoid(z[96:])
        bvq_ref[0, :, _S2 * t:_S2 * (t + 1)] = g

        scores = jnp.dot(cb, g, preferred_element_type=_F32)
        fsq = jnp.sum(g * g, axis=0, keepdims=True)
        dist = (fsq - 2.0 * scores) + cbsq
        minval = jnp.min(dist, axis=0, keepdims=True)
        kiota = jax.lax.broadcasted_iota(jnp.int32, dist.shape, 0)
        idx = jnp.min(jnp.where(dist == minval, kiota, 1024), axis=0,
                      keepdims=True)
        idx_ref[0, :, _S2 * t:_S2 * (t + 1)] = idx

        onehot = (kiota == idx).astype(_F32)
        quant = jnp.dot(cbt_ref[...], onehot, preferred_element_type=_F32)
        lat_ref[0, :, _S2 * t:_S2 * (t + 1)] = g + (quant - g)


def kernel(x, conv1_w, conv1_b, rw1_w, rw1_b, conv2_w, conv2_b, rw2_w, rw2_b,
           codebook):
    B = x.shape[0]
    N1 = 128 * 256
    N2 = 32 * 256

    xp = jnp.pad(x, ((0, 0), (0, 0), (2, 2), (0, 0)))
    p1 = jnp.stack([xp[:, :, kh:kh + 509:4, :] for kh in range(8)], axis=1)
    p1 = p1[:, :, :, _H1_ORDER, :].reshape(B, 16, N1)
    w1m = jnp.transpose(conv1_w[:, :, :, 0], (0, 2, 1)).reshape(48, 16)
    w2m = jnp.transpose(conv2_w[:, :, :, 0], (0, 2, 1)).reshape(96, 384)

    full = lambda *s: pl.BlockSpec(s, lambda b: tuple(0 for _ in s))
    bvq, lat, idx = pl.pallas_call(
        _fused_kernel,
        grid=(B,),
        in_specs=[
            pl.BlockSpec((1, 16, N1), lambda b: (b, 0, 0)),
            full(48, 16), full(48, 1), full(96, 48), full(96, 1),
            full(96, 384), full(96, 1), full(192, 96), full(192, 1),
            full(1024, 96), full(96, 1024),
        ],
        out_specs=[
            pl.BlockSpec((1, 96, N2), lambda b: (b, 0, 0)),
            pl.BlockSpec((1, 96, N2), lambda b: (b, 0, 0)),
            pl.BlockSpec((1, 1, N2), lambda b: (b, 0, 0)),
        ],
        out_shape=[
            jax.ShapeDtypeStruct((B, 96, N2), _F32),
            jax.ShapeDtypeStruct((B, 96, N2), _F32),
            jax.ShapeDtypeStruct((B, 1, N2), jnp.int32),
        ],
        scratch_shapes=[pltpu.VMEM((48, 4 * _SEC), _F32)],
    )(p1, w1m, conv1_b[:, None], rw1_w[:, :, 0, 0], rw1_b[:, None],
      w2m, conv2_b[:, None], rw2_w[:, :, 0, 0], rw2_b[:, None],
      codebook, jnp.transpose(codebook))

    latent = lat.reshape(B, 96, 32, 256)
    beforvq = bvq.reshape(B, 96, 32, 256)
    indices = idx.reshape(B, N2)
    return (latent, indices, beforvq)
